# baseline (speedup 1.0000x reference)
import jax
import jax.numpy as jnp
from jax import lax
from jax.experimental import pallas as pl
from jax.experimental.pallas import tpu as pltpu
from jax.experimental.pallas import tpu_sc as plsc

_B = 16384
_D = 64
_NC = 2
_NS = 16
_NW = _NC * _NS
_RPW = _B // _NW
_IDX_CHUNK = 128
_NCHUNK = _RPW // _IDX_CHUNK
_LANES = 16


def _body(idx_hbm, src_hbm, trg_hbm, table_hbm, scores_hbm, parts_hbm,
          idx_v, rel_v, src_v, trg_v, scores_v, sq_v, pacc_v, sem):
    wid = lax.axis_index("s") * _NC + lax.axis_index("c")
    base = wid * _RPW

    pltpu.sync_copy(idx_hbm.at[wid], idx_v)
    copies = []
    for j in range(_NCHUNK):
        copies.append(
            pltpu.async_copy(table_hbm.at[idx_v.at[j]],
                             rel_v.at[pl.ds(j * _IDX_CHUNK, _IDX_CHUNK)], sem))
    copies.append(pltpu.async_copy(src_hbm.at[pl.ds(base, _RPW)], src_v, sem))
    copies.append(pltpu.async_copy(trg_hbm.at[pl.ds(base, _RPW)], trg_v, sem))
    for c in copies:
        c.wait()

    lane = lax.iota(jnp.int32, _LANES)

    def group(g, acc_sq):
        rbase = g * _LANES
        for i in range(_LANES):
            r = rbase + i
            acc = None
            for c in range(_D // _LANES):
                sl = pl.ds(c * _LANES, _LANES)
                s = src_v[r, sl]
                e = rel_v[r, sl]
                t = trg_v[r, sl]
                p = s * e * t
                acc = p if acc is None else acc + p
                acc_sq = acc_sq + (s * s + e * e + t * t)
            pacc_v[i, pl.ds(0, _LANES)] = acc
        svec = None
        for c in range(_LANES):
            t = plsc.load_gather(
                pacc_v, [lane, jnp.full((_LANES,), c, jnp.int32)])
            svec = t if svec is None else svec + t
        scores_v[pl.ds(rbase, _LANES)] = svec
        return acc_sq

    acc_sq = lax.fori_loop(0, _RPW // _LANES, group,
                           jnp.zeros((_LANES,), jnp.float32))
    sq_v[...] = acc_sq
    pltpu.sync_copy(scores_v, scores_hbm.at[pl.ds(base, _RPW)])
    pltpu.sync_copy(sq_v, parts_hbm.at[wid])


@jax.jit
def _score(idx, src, trg, table):
    mesh = plsc.VectorSubcoreMesh(core_axis_name="c", subcore_axis_name="s",
                                  num_cores=_NC, num_subcores=_NS)
    f = pl.kernel(
        _body,
        out_type=(jax.ShapeDtypeStruct((_B,), jnp.float32),
                  jax.ShapeDtypeStruct((_NW, _LANES), jnp.float32)),
        mesh=mesh,
        compiler_params=pltpu.CompilerParams(needs_layout_passes=False,
                                             use_tc_tiling_on_sc=False),
        scratch_types=[
            pltpu.VMEM((_NCHUNK, _IDX_CHUNK), jnp.int32),
            pltpu.VMEM((_RPW, _D), jnp.float32),
            pltpu.VMEM((_RPW, _D), jnp.float32),
            pltpu.VMEM((_RPW, _D), jnp.float32),
            pltpu.VMEM((_RPW,), jnp.float32),
            pltpu.VMEM((_LANES,), jnp.float32),
            pltpu.VMEM((_LANES, _LANES + 1), jnp.float32),
            pltpu.SemaphoreType.DMA,
        ],
    )
    return f(idx, src, trg, table)


def kernel(src_node_embs, trg_node_embs, rel_ids, relation_embeddings):
    idx = rel_ids.astype(jnp.int32).reshape(_NW, _NCHUNK, _IDX_CHUNK)
    scores, parts = _score(idx, src_node_embs, trg_node_embs,
                           relation_embeddings)
    reg = jnp.sum(parts) * (1.0 / (_B * _D))
    return scores, reg
